# baseline (speedup 1.0000x reference)
import functools

import jax
import jax.numpy as jnp
from jax import lax
from jax.experimental import pallas as pl
from jax.experimental.pallas import tpu as pltpu
from jax.experimental.pallas import tpu_sc as plsc

_N = 10000
_NP = 10112
_K = 128
_NC = 2
_NS = 16
_TILES = _NC * _NS
_RPT = _NP // _NS
_EPS = 1e-5


def _sc_mesh():
    return plsc.VectorSubcoreMesh(
        core_axis_name="c", subcore_axis_name="s",
        num_cores=_NC, num_subcores=_NS)


_SC_PARAMS = pltpu.CompilerParams(use_tc_tiling_on_sc=False)


@functools.cache
def _deg_kernel(nch):
    @functools.partial(
        pl.kernel,
        out_type=jax.ShapeDtypeStruct((_NC * _NP, 1), jnp.float32),
        mesh=_sc_mesh(),
        compiler_params=_SC_PARAMS,
        scratch_types=[
            pltpu.VMEM((nch, _K), jnp.int32),
            pltpu.VMEM((_K, 1), jnp.float32),
            pltpu.VMEM_SHARED((_NP, 1), jnp.float32),
        ],
    )
    def body(dst_hbm, ones_hbm, zeros_hbm, out_hbm, dst_v, ones_v, acc_sh):
        cid = lax.axis_index("c")
        sid = lax.axis_index("s")
        t = cid * _NS + sid
        row0 = sid * _RPT
        pltpu.sync_copy(zeros_hbm.at[pl.ds(row0, _RPT)], acc_sh.at[pl.ds(row0, _RPT)])
        pltpu.sync_copy(dst_hbm.at[t], dst_v)
        pltpu.sync_copy(ones_hbm, ones_v)
        plsc.subcore_barrier()

        def step(j, carry):
            pltpu.sync_copy(ones_v, acc_sh.at[dst_v.at[j]], add=True)
            return carry

        lax.fori_loop(0, nch, step, 0)
        plsc.subcore_barrier()
        pltpu.sync_copy(acc_sh.at[pl.ds(row0, _RPT)],
                        out_hbm.at[pl.ds(cid * _NP + row0, _RPT)])

    return body


@functools.cache
def _spmm_kernel(nch, dh, stage_g):
    @functools.partial(
        pl.kernel,
        out_type=jax.ShapeDtypeStruct((_NC * _NP, dh), jnp.float32),
        mesh=_sc_mesh(),
        compiler_params=_SC_PARAMS,
        scratch_types=[
            pltpu.VMEM((nch, _K), jnp.int32),
            pltpu.VMEM((nch, _K), jnp.int32),
            pltpu.VMEM((_K, dh), jnp.float32),
            pltpu.VMEM_SHARED((_NP, dh), jnp.float32),
        ] + ([pltpu.VMEM_SHARED((_NP, dh), jnp.float32)] if stage_g else []),
    )
    def body(g_hbm, src_hbm, dst_hbm, zeros_hbm, out_hbm,
             src_v, dst_v, rows_v, acc_sh, *maybe_g_sh):
        cid = lax.axis_index("c")
        sid = lax.axis_index("s")
        t = cid * _NS + sid
        row0 = sid * _RPT
        pltpu.sync_copy(zeros_hbm.at[pl.ds(row0, _RPT)], acc_sh.at[pl.ds(row0, _RPT)])
        g_ref = maybe_g_sh[0] if stage_g else g_hbm
        if stage_g:
            pltpu.sync_copy(g_hbm.at[pl.ds(row0, _RPT)], g_ref.at[pl.ds(row0, _RPT)])
        pltpu.sync_copy(src_hbm.at[t], src_v)
        pltpu.sync_copy(dst_hbm.at[t], dst_v)
        plsc.subcore_barrier()

        def step(j, carry):
            pltpu.sync_copy(g_ref.at[src_v.at[j]], rows_v)
            pltpu.sync_copy(rows_v, acc_sh.at[dst_v.at[j]], add=True)
            return carry

        lax.fori_loop(0, nch, step, 0)
        plsc.subcore_barrier()
        pltpu.sync_copy(acc_sh.at[pl.ds(row0, _RPT)],
                        out_hbm.at[pl.ds(cid * _NP + row0, _RPT)])

    return body


def _tc1a_body(x, w1, h1_o):
    h1_o[...] = jnp.dot(x[...], w1[...], preferred_element_type=jnp.float32)


def _tc1b_body(degp, h1, dis_o, g1_o):
    deg = degp[:_NP, 0:1] + degp[_NP:, 0:1] + 1.0
    dis = lax.rsqrt(deg)
    dis_o[...] = dis
    g1_o[...] = h1[...] * dis


def _tc2_body(accp, g1, dis, w2, b1, x1_o, g2_o):
    s1 = accp[:_NP] + accp[_NP:] + g1[...]
    x1 = dis[...] * s1 + b1[...]
    rows = lax.broadcasted_iota(jnp.int32, (_NP, 1), 0)
    x1 = jnp.where(rows < _N, x1, 0.0)
    h2 = jnp.dot(x1, w2[...], preferred_element_type=jnp.float32)
    x1_o[...] = x1
    g2_o[...] = h2 * dis[...]


def _dense_bn(h, w, b, g, be, ones_row):
    hl = jnp.maximum(jnp.dot(h, w, preferred_element_type=jnp.float32) + b, 0.0)
    sums = jnp.dot(ones_row, hl, preferred_element_type=jnp.float32)
    sqs = jnp.dot(ones_row, hl * hl, preferred_element_type=jnp.float32)
    mu = sums * (1.0 / _N)
    var = sqs * (1.0 / _N) - mu * mu
    return g * (hl - mu) * lax.rsqrt(var + _EPS) + be


def _tc3_body(accp, g2, x1, dis, b2,
              wl, bl, gl, bel, wm1, bm1, gm1, bem1, wm2, bm2, gm2, bem2, out_o):
    x2 = dis[...] * (accp[:_NP] + accp[_NP:] + g2[...]) + b2[...]
    h = jnp.concatenate([x1[:_N], x2[:_N]], axis=1)
    ones_row = jnp.ones((1, _N), jnp.float32)
    h = _dense_bn(h, wl[...], bl[...], gl[...], bel[...], ones_row)
    h = _dense_bn(h, wm1[...], bm1[...], gm1[...], bem1[...], ones_row)
    h = _dense_bn(h, wm2[...], bm2[...], gm2[...], bem2[...], ones_row)
    out_o[...] = h


def kernel(x, edge_index, batch, W1, b1, W2, b2, Wl, bl, gl, bel,
           Wm1, bm1, gm1, bem1, Wm2, bm2, gm2, bem2):
    del batch
    e = edge_index.shape[1]
    nch = -(-e // (_TILES * _K))
    pad = _TILES * _K * nch - e
    f32 = jnp.float32

    spread = _N + (jnp.arange(pad, dtype=jnp.int32) % (_NP - _N))
    src = jnp.concatenate([edge_index[0], spread])
    dst = jnp.concatenate([edge_index[1], spread])
    srcp = src.reshape(_TILES, nch, _K)
    dstp = dst.reshape(_TILES, nch, _K)
    xp = jnp.pad(x, ((0, _NP - _N), (0, 0)))

    degp = _deg_kernel(nch)(
        dstp, jnp.ones((_K, 1), f32), jnp.zeros((_NP, 1), f32))

    h1 = pl.pallas_call(
        _tc1a_body, out_shape=jax.ShapeDtypeStruct((_NP, 32), f32))(xp, W1)
    dis, g1 = pl.pallas_call(
        _tc1b_body,
        out_shape=[jax.ShapeDtypeStruct((_NP, 1), f32),
                   jax.ShapeDtypeStruct((_NP, 32), f32)],
    )(degp, h1)

    accp1 = _spmm_kernel(nch, 32, True)(g1, srcp, dstp, jnp.zeros((_NP, 32), f32))

    x1, g2 = pl.pallas_call(
        _tc2_body,
        out_shape=[jax.ShapeDtypeStruct((_NP, 32), f32),
                   jax.ShapeDtypeStruct((_NP, 64), f32)],
    )(accp1, g1, dis, W2, b1.reshape(1, -1))

    accp2 = _spmm_kernel(nch, 64, True)(g2, srcp, dstp, jnp.zeros((_NP, 64), f32))

    out = pl.pallas_call(
        _tc3_body,
        out_shape=jax.ShapeDtypeStruct((_N, 10), f32),
    )(accp2, g2, x1, dis, b2.reshape(1, -1),
      Wl, bl.reshape(1, -1), gl.reshape(1, -1), bel.reshape(1, -1),
      Wm1, bm1.reshape(1, -1), gm1.reshape(1, -1), bem1.reshape(1, -1),
      Wm2, bm2.reshape(1, -1), gm2.reshape(1, -1), bem2.reshape(1, -1))
    return out


# speedup vs baseline: 1.0231x; 1.0231x over previous
import functools

import jax
import jax.numpy as jnp
from jax import lax
from jax.experimental import pallas as pl
from jax.experimental.pallas import tpu as pltpu
from jax.experimental.pallas import tpu_sc as plsc

_N = 10000
_NP = 10112
_K = 128
_NC = 2
_NS = 16
_TILES = _NC * _NS
_RPT = _NP // _NS
_EPS = 1e-5


def _sc_mesh():
    return plsc.VectorSubcoreMesh(
        core_axis_name="c", subcore_axis_name="s",
        num_cores=_NC, num_subcores=_NS)


_SC_PARAMS = pltpu.CompilerParams(use_tc_tiling_on_sc=False)


@functools.cache
def _deg_kernel(nch):
    @functools.partial(
        pl.kernel,
        out_type=jax.ShapeDtypeStruct((_NC * _NP, 8), jnp.float32),
        mesh=_sc_mesh(),
        compiler_params=_SC_PARAMS,
        scratch_types=[
            pltpu.VMEM((nch, _K), jnp.int32),
            pltpu.VMEM((_K, 8), jnp.float32),
            pltpu.VMEM_SHARED((_NP, 8), jnp.float32),
        ],
    )
    def body(dst_hbm, ones_hbm, zeros_hbm, out_hbm, dst_v, ones_v, acc_sh):
        cid = lax.axis_index("c")
        sid = lax.axis_index("s")
        t = cid * _NS + sid
        row0 = sid * _RPT
        pltpu.sync_copy(zeros_hbm.at[pl.ds(row0, _RPT)], acc_sh.at[pl.ds(row0, _RPT)])
        pltpu.sync_copy(dst_hbm.at[t], dst_v)
        pltpu.sync_copy(ones_hbm, ones_v)
        plsc.subcore_barrier()

        def step(j, carry):
            pltpu.sync_copy(ones_v, acc_sh.at[dst_v.at[j]], add=True)
            return carry

        lax.fori_loop(0, nch, step, 0)
        plsc.subcore_barrier()
        pltpu.sync_copy(acc_sh.at[pl.ds(row0, _RPT)],
                        out_hbm.at[pl.ds(cid * _NP + row0, _RPT)])

    return body


@functools.cache
def _spmm_kernel(nch, dh, stage_g):
    @functools.partial(
        pl.kernel,
        out_type=jax.ShapeDtypeStruct((_NC * _NP, dh), jnp.float32),
        mesh=_sc_mesh(),
        compiler_params=_SC_PARAMS,
        scratch_types=[
            pltpu.VMEM((nch, _K), jnp.int32),
            pltpu.VMEM((nch, _K), jnp.int32),
            pltpu.VMEM((_K, dh), jnp.float32),
            pltpu.VMEM_SHARED((_NP, dh), jnp.float32),
        ] + ([pltpu.VMEM_SHARED((_NP, dh), jnp.float32)] if stage_g else []),
    )
    def body(g_hbm, src_hbm, dst_hbm, zeros_hbm, out_hbm,
             src_v, dst_v, rows_v, acc_sh, *maybe_g_sh):
        cid = lax.axis_index("c")
        sid = lax.axis_index("s")
        t = cid * _NS + sid
        row0 = sid * _RPT
        pltpu.sync_copy(zeros_hbm.at[pl.ds(row0, _RPT)], acc_sh.at[pl.ds(row0, _RPT)])
        g_ref = maybe_g_sh[0] if stage_g else g_hbm
        if stage_g:
            pltpu.sync_copy(g_hbm.at[pl.ds(row0, _RPT)], g_ref.at[pl.ds(row0, _RPT)])
        pltpu.sync_copy(src_hbm.at[t], src_v)
        pltpu.sync_copy(dst_hbm.at[t], dst_v)
        plsc.subcore_barrier()

        def step(j, carry):
            pltpu.sync_copy(g_ref.at[src_v.at[j]], rows_v)
            pltpu.sync_copy(rows_v, acc_sh.at[dst_v.at[j]], add=True)
            return carry

        lax.fori_loop(0, nch, step, 0)
        plsc.subcore_barrier()
        pltpu.sync_copy(acc_sh.at[pl.ds(row0, _RPT)],
                        out_hbm.at[pl.ds(cid * _NP + row0, _RPT)])

    return body


def _tc1a_body(x, w1, h1_o):
    h1_o[...] = jnp.dot(x[...], w1[...], preferred_element_type=jnp.float32)


def _tc1b_body(degp, h1, dis_o, g1_o):
    deg = degp[:_NP, 0:1] + degp[_NP:, 0:1] + 1.0
    dis = lax.rsqrt(deg)
    dis_o[...] = dis
    g1_o[...] = h1[...] * dis


def _tc2_body(accp, g1, dis, w2, b1, x1_o, g2_o):
    s1 = accp[:_NP] + accp[_NP:] + g1[...]
    x1 = dis[...] * s1 + b1[...]
    rows = lax.broadcasted_iota(jnp.int32, (_NP, 1), 0)
    x1 = jnp.where(rows < _N, x1, 0.0)
    h2 = jnp.dot(x1, w2[...], preferred_element_type=jnp.float32)
    x1_o[...] = x1
    g2_o[...] = h2 * dis[...]


def _dense_bn(h, w, b, g, be, ones_row):
    hl = jnp.maximum(jnp.dot(h, w, preferred_element_type=jnp.float32) + b, 0.0)
    sums = jnp.dot(ones_row, hl, preferred_element_type=jnp.float32)
    sqs = jnp.dot(ones_row, hl * hl, preferred_element_type=jnp.float32)
    mu = sums * (1.0 / _N)
    var = sqs * (1.0 / _N) - mu * mu
    return g * (hl - mu) * lax.rsqrt(var + _EPS) + be


def _tc3_body(accp, g2, x1, dis, b2,
              wl, bl, gl, bel, wm1, bm1, gm1, bem1, wm2, bm2, gm2, bem2, out_o):
    x2 = dis[...] * (accp[:_NP] + accp[_NP:] + g2[...]) + b2[...]
    h = jnp.concatenate([x1[:_N], x2[:_N]], axis=1)
    ones_row = jnp.ones((1, _N), jnp.float32)
    h = _dense_bn(h, wl[...], bl[...], gl[...], bel[...], ones_row)
    h = _dense_bn(h, wm1[...], bm1[...], gm1[...], bem1[...], ones_row)
    h = _dense_bn(h, wm2[...], bm2[...], gm2[...], bem2[...], ones_row)
    out_o[...] = h


def kernel(x, edge_index, batch, W1, b1, W2, b2, Wl, bl, gl, bel,
           Wm1, bm1, gm1, bem1, Wm2, bm2, gm2, bem2):
    del batch
    e = edge_index.shape[1]
    nch = -(-e // (_TILES * _K))
    pad = _TILES * _K * nch - e
    f32 = jnp.float32

    spread = _N + (jnp.arange(pad, dtype=jnp.int32) % (_NP - _N))
    src = jnp.concatenate([edge_index[0], spread])
    dst = jnp.concatenate([edge_index[1], spread])
    srcp = src.reshape(_TILES, nch, _K)
    dstp = dst.reshape(_TILES, nch, _K)
    xp = jnp.pad(x, ((0, _NP - _N), (0, 0)))

    degp = _deg_kernel(nch)(
        dstp, jnp.ones((_K, 8), f32), jnp.zeros((_NP, 8), f32))

    h1 = pl.pallas_call(
        _tc1a_body, out_shape=jax.ShapeDtypeStruct((_NP, 32), f32))(xp, W1)
    dis, g1 = pl.pallas_call(
        _tc1b_body,
        out_shape=[jax.ShapeDtypeStruct((_NP, 1), f32),
                   jax.ShapeDtypeStruct((_NP, 32), f32)],
    )(degp, h1)

    accp1 = _spmm_kernel(nch, 32, True)(g1, srcp, dstp, jnp.zeros((_NP, 32), f32))

    x1, g2 = pl.pallas_call(
        _tc2_body,
        out_shape=[jax.ShapeDtypeStruct((_NP, 32), f32),
                   jax.ShapeDtypeStruct((_NP, 64), f32)],
    )(accp1, g1, dis, W2, b1.reshape(1, -1))

    accp2 = _spmm_kernel(nch, 64, True)(g2, srcp, dstp, jnp.zeros((_NP, 64), f32))

    out = pl.pallas_call(
        _tc3_body,
        out_shape=jax.ShapeDtypeStruct((_N, 10), f32),
    )(accp2, g2, x1, dis, b2.reshape(1, -1),
      Wl, bl.reshape(1, -1), gl.reshape(1, -1), bel.reshape(1, -1),
      Wm1, bm1.reshape(1, -1), gm1.reshape(1, -1), bem1.reshape(1, -1),
      Wm2, bm2.reshape(1, -1), gm2.reshape(1, -1), bem2.reshape(1, -1))
    return out


# speedup vs baseline: 1.0302x; 1.0070x over previous
import functools

import numpy as np

import jax
import jax.numpy as jnp
from jax import lax
from jax.experimental import pallas as pl
from jax.experimental.pallas import tpu as pltpu
from jax.experimental.pallas import tpu_sc as plsc

_N = 10000
_NP = 10112
_K = 128
_NC = 2
_NS = 16
_TILES = _NC * _NS
_RPT = _NP // _NS
_EPS = 1e-5


def _sc_mesh():
    return plsc.VectorSubcoreMesh(
        core_axis_name="c", subcore_axis_name="s",
        num_cores=_NC, num_subcores=_NS)


_SC_PARAMS = pltpu.CompilerParams(use_tc_tiling_on_sc=False)


@functools.cache
def _deg_kernel(nch):
    @functools.partial(
        pl.kernel,
        out_type=jax.ShapeDtypeStruct((_NC * _NP, 8), jnp.float32),
        mesh=_sc_mesh(),
        compiler_params=_SC_PARAMS,
        scratch_types=[
            pltpu.VMEM((nch, _K), jnp.int32),
            pltpu.VMEM((_K, 8), jnp.float32),
            pltpu.VMEM_SHARED((_NP, 8), jnp.float32),
        ],
    )
    def body(dst_hbm, ones_hbm, zeros_hbm, out_hbm, dst_v, ones_v, acc_sh):
        cid = lax.axis_index("c")
        sid = lax.axis_index("s")
        t = cid * _NS + sid
        row0 = sid * _RPT
        pltpu.sync_copy(zeros_hbm.at[pl.ds(row0, _RPT)], acc_sh.at[pl.ds(row0, _RPT)])
        pltpu.sync_copy(dst_hbm.at[t], dst_v)
        pltpu.sync_copy(ones_hbm, ones_v)
        plsc.subcore_barrier()

        def step(j, carry):
            pltpu.sync_copy(ones_v, acc_sh.at[dst_v.at[j]], add=True)
            return carry

        lax.fori_loop(0, nch, step, 0)
        plsc.subcore_barrier()
        pltpu.sync_copy(acc_sh.at[pl.ds(row0, _RPT)],
                        out_hbm.at[pl.ds(cid * _NP + row0, _RPT)])

    return body


@functools.cache
def _spmm_kernel(nch, dh, stage_g):
    @functools.partial(
        pl.kernel,
        out_type=jax.ShapeDtypeStruct((_NC * _NP, dh), jnp.float32),
        mesh=_sc_mesh(),
        compiler_params=_SC_PARAMS,
        scratch_types=[
            pltpu.VMEM((nch, _K), jnp.int32),
            pltpu.VMEM((nch, _K), jnp.int32),
            pltpu.VMEM((_K, dh), jnp.float32),
            pltpu.VMEM_SHARED((_NP, dh), jnp.float32),
        ] + ([pltpu.VMEM_SHARED((_NP, dh), jnp.float32)] if stage_g else []),
    )
    def body(g_hbm, src_hbm, dst_hbm, zeros_hbm, out_hbm,
             src_v, dst_v, rows_v, acc_sh, *maybe_g_sh):
        cid = lax.axis_index("c")
        sid = lax.axis_index("s")
        t = cid * _NS + sid
        row0 = sid * _RPT
        pltpu.sync_copy(zeros_hbm.at[pl.ds(row0, _RPT)], acc_sh.at[pl.ds(row0, _RPT)])
        g_ref = maybe_g_sh[0] if stage_g else g_hbm
        if stage_g:
            pltpu.sync_copy(g_hbm.at[pl.ds(row0, _RPT)], g_ref.at[pl.ds(row0, _RPT)])
        pltpu.sync_copy(src_hbm.at[t], src_v)
        pltpu.sync_copy(dst_hbm.at[t], dst_v)
        plsc.subcore_barrier()

        def step(j, carry):
            pltpu.sync_copy(g_ref.at[src_v.at[j]], rows_v)
            pltpu.sync_copy(rows_v, acc_sh.at[dst_v.at[j]], add=True)
            return carry

        lax.fori_loop(0, nch, step, 0)
        plsc.subcore_barrier()
        pltpu.sync_copy(acc_sh.at[pl.ds(row0, _RPT)],
                        out_hbm.at[pl.ds(cid * _NP + row0, _RPT)])

    return body


def _tc1a_body(x, w1, h1_o):
    h1_o[...] = jnp.dot(x[...], w1[...], preferred_element_type=jnp.float32)


def _tc1b_body(degp, h1, dis_o, g1_o):
    deg = degp[:_NP, 0:1] + degp[_NP:, 0:1] + 1.0
    dis = lax.rsqrt(deg)
    dis_o[...] = dis
    g1_o[...] = h1[...] * dis


def _tc2_body(accp, g1, dis, w2, b1, x1_o, g2_o):
    s1 = accp[:_NP] + accp[_NP:] + g1[...]
    x1 = dis[...] * s1 + b1[...]
    rows = lax.broadcasted_iota(jnp.int32, (_NP, 1), 0)
    x1 = jnp.where(rows < _N, x1, 0.0)
    h2 = jnp.dot(x1, w2[...], preferred_element_type=jnp.float32)
    x1_o[...] = x1
    g2_o[...] = h2 * dis[...]


def _dense_bn(h, w, b, g, be, ones_row):
    hl = jnp.maximum(jnp.dot(h, w, preferred_element_type=jnp.float32) + b, 0.0)
    sums = jnp.dot(ones_row, hl, preferred_element_type=jnp.float32)
    sqs = jnp.dot(ones_row, hl * hl, preferred_element_type=jnp.float32)
    mu = sums * (1.0 / _N)
    var = sqs * (1.0 / _N) - mu * mu
    return g * (hl - mu) * lax.rsqrt(var + _EPS) + be


def _tc3_body(accp, g2, x1, dis, b2,
              wl, bl, gl, bel, wm1, bm1, gm1, bem1, wm2, bm2, gm2, bem2, out_o):
    x2 = dis[...] * (accp[:_NP] + accp[_NP:] + g2[...]) + b2[...]
    h = jnp.concatenate([x1[:_N], x2[:_N]], axis=1)
    ones_row = jnp.ones((1, _N), jnp.float32)
    h = _dense_bn(h, wl[...], bl[...], gl[...], bel[...], ones_row)
    h = _dense_bn(h, wm1[...], bm1[...], gm1[...], bem1[...], ones_row)
    h = _dense_bn(h, wm2[...], bm2[...], gm2[...], bem2[...], ones_row)
    out_o[...] = h


def kernel(x, edge_index, batch, W1, b1, W2, b2, Wl, bl, gl, bel,
           Wm1, bm1, gm1, bem1, Wm2, bm2, gm2, bem2):
    del batch
    e = edge_index.shape[1]
    nch = -(-e // (_TILES * _K))
    pad = _TILES * _K * nch - e
    f32 = jnp.float32

    spread = jnp.asarray(
        np.broadcast_to(_N + (np.arange(pad, dtype=np.int32) % (_NP - _N)),
                        (2, pad)))
    both = jnp.concatenate([edge_index, spread], axis=1)
    slabs = both.reshape(2, _TILES, nch, _K)
    srcp = slabs[0]
    dstp = slabs[1]
    xp = jnp.pad(x, ((0, _NP - _N), (0, 0)))

    degp = _deg_kernel(nch)(
        dstp, jnp.ones((_K, 8), f32), jnp.zeros((_NP, 8), f32))

    h1 = pl.pallas_call(
        _tc1a_body, out_shape=jax.ShapeDtypeStruct((_NP, 32), f32))(xp, W1)
    dis, g1 = pl.pallas_call(
        _tc1b_body,
        out_shape=[jax.ShapeDtypeStruct((_NP, 1), f32),
                   jax.ShapeDtypeStruct((_NP, 32), f32)],
    )(degp, h1)

    accp1 = _spmm_kernel(nch, 32, True)(g1, srcp, dstp, jnp.zeros((_NP, 32), f32))

    x1, g2 = pl.pallas_call(
        _tc2_body,
        out_shape=[jax.ShapeDtypeStruct((_NP, 32), f32),
                   jax.ShapeDtypeStruct((_NP, 64), f32)],
    )(accp1, g1, dis, W2, b1.reshape(1, -1))

    accp2 = _spmm_kernel(nch, 64, True)(g2, srcp, dstp, jnp.zeros((_NP, 64), f32))

    out = pl.pallas_call(
        _tc3_body,
        out_shape=jax.ShapeDtypeStruct((_N, 10), f32),
    )(accp2, g2, x1, dis, b2.reshape(1, -1),
      Wl, bl.reshape(1, -1), gl.reshape(1, -1), bel.reshape(1, -1),
      Wm1, bm1.reshape(1, -1), gm1.reshape(1, -1), bem1.reshape(1, -1),
      Wm2, bm2.reshape(1, -1), gm2.reshape(1, -1), bem2.reshape(1, -1))
    return out


# speedup vs baseline: 1.0502x; 1.0194x over previous
import functools

import numpy as np

import jax
import jax.numpy as jnp
from jax import lax
from jax.experimental import pallas as pl
from jax.experimental.pallas import tpu as pltpu
from jax.experimental.pallas import tpu_sc as plsc

_N = 10000
_NP = 10112
_K = 256
_NC = 2
_NS = 16
_TILES = _NC * _NS
_RPT = _NP // _NS
_EPS = 1e-5


def _sc_mesh():
    return plsc.VectorSubcoreMesh(
        core_axis_name="c", subcore_axis_name="s",
        num_cores=_NC, num_subcores=_NS)


_SC_PARAMS = pltpu.CompilerParams(use_tc_tiling_on_sc=False)


@functools.cache
def _deg_kernel(nch):
    @functools.partial(
        pl.kernel,
        out_type=jax.ShapeDtypeStruct((_NC * _NP, 8), jnp.float32),
        mesh=_sc_mesh(),
        compiler_params=_SC_PARAMS,
        scratch_types=[
            pltpu.VMEM((nch, _K), jnp.int32),
            pltpu.VMEM((_K, 8), jnp.float32),
            pltpu.VMEM_SHARED((_NP, 8), jnp.float32),
        ],
    )
    def body(dst_hbm, ones_hbm, zeros_hbm, out_hbm, dst_v, ones_v, acc_sh):
        cid = lax.axis_index("c")
        sid = lax.axis_index("s")
        t = cid * _NS + sid
        row0 = sid * _RPT
        pltpu.sync_copy(zeros_hbm.at[pl.ds(row0, _RPT)], acc_sh.at[pl.ds(row0, _RPT)])
        pltpu.sync_copy(dst_hbm.at[t], dst_v)
        pltpu.sync_copy(ones_hbm, ones_v)
        plsc.subcore_barrier()

        def step(j, carry):
            pltpu.sync_copy(ones_v, acc_sh.at[dst_v.at[j]], add=True)
            return carry

        lax.fori_loop(0, nch, step, 0)
        plsc.subcore_barrier()
        pltpu.sync_copy(acc_sh.at[pl.ds(row0, _RPT)],
                        out_hbm.at[pl.ds(cid * _NP + row0, _RPT)])

    return body


@functools.cache
def _spmm_kernel(nch, dh, stage_g):
    @functools.partial(
        pl.kernel,
        out_type=jax.ShapeDtypeStruct((_NC * _NP, dh), jnp.float32),
        mesh=_sc_mesh(),
        compiler_params=_SC_PARAMS,
        scratch_types=[
            pltpu.VMEM((nch, _K), jnp.int32),
            pltpu.VMEM((nch, _K), jnp.int32),
            pltpu.VMEM((_K, dh), jnp.float32),
            pltpu.VMEM_SHARED((_NP, dh), jnp.float32),
        ] + ([pltpu.VMEM_SHARED((_NP, dh), jnp.float32)] if stage_g else []),
    )
    def body(g_hbm, src_hbm, dst_hbm, zeros_hbm, out_hbm,
             src_v, dst_v, rows_v, acc_sh, *maybe_g_sh):
        cid = lax.axis_index("c")
        sid = lax.axis_index("s")
        t = cid * _NS + sid
        row0 = sid * _RPT
        pltpu.sync_copy(zeros_hbm.at[pl.ds(row0, _RPT)], acc_sh.at[pl.ds(row0, _RPT)])
        g_ref = maybe_g_sh[0] if stage_g else g_hbm
        if stage_g:
            pltpu.sync_copy(g_hbm.at[pl.ds(row0, _RPT)], g_ref.at[pl.ds(row0, _RPT)])
        pltpu.sync_copy(src_hbm.at[t], src_v)
        pltpu.sync_copy(dst_hbm.at[t], dst_v)
        plsc.subcore_barrier()

        def step(j, carry):
            pltpu.sync_copy(g_ref.at[src_v.at[j]], rows_v)
            pltpu.sync_copy(rows_v, acc_sh.at[dst_v.at[j]], add=True)
            return carry

        lax.fori_loop(0, nch, step, 0)
        plsc.subcore_barrier()
        pltpu.sync_copy(acc_sh.at[pl.ds(row0, _RPT)],
                        out_hbm.at[pl.ds(cid * _NP + row0, _RPT)])

    return body


def _tc1a_body(x, w1, h1_o):
    h1_o[...] = jnp.dot(x[...], w1[...], preferred_element_type=jnp.float32)


def _tc1b_body(degp, h1, dis_o, g1_o):
    deg = degp[:_NP, 0:1] + degp[_NP:, 0:1] + 1.0
    dis = lax.rsqrt(deg)
    dis_o[...] = dis
    g1_o[...] = h1[...] * dis


def _tc2_body(accp, g1, dis, w2, b1, x1_o, g2_o):
    s1 = accp[:_NP] + accp[_NP:] + g1[...]
    x1 = dis[...] * s1 + b1[...]
    rows = lax.broadcasted_iota(jnp.int32, (_NP, 1), 0)
    x1 = jnp.where(rows < _N, x1, 0.0)
    h2 = jnp.dot(x1, w2[...], preferred_element_type=jnp.float32)
    x1_o[...] = x1
    g2_o[...] = h2 * dis[...]


def _dense_bn(h, w, b, g, be, ones_row):
    hl = jnp.maximum(jnp.dot(h, w, preferred_element_type=jnp.float32) + b, 0.0)
    sums = jnp.dot(ones_row, hl, preferred_element_type=jnp.float32)
    sqs = jnp.dot(ones_row, hl * hl, preferred_element_type=jnp.float32)
    mu = sums * (1.0 / _N)
    var = sqs * (1.0 / _N) - mu * mu
    return g * (hl - mu) * lax.rsqrt(var + _EPS) + be


def _tc3_body(accp, g2, x1, dis, b2,
              wl, bl, gl, bel, wm1, bm1, gm1, bem1, wm2, bm2, gm2, bem2, out_o):
    x2 = dis[...] * (accp[:_NP] + accp[_NP:] + g2[...]) + b2[...]
    h = jnp.concatenate([x1[:_N], x2[:_N]], axis=1)
    ones_row = jnp.ones((1, _N), jnp.float32)
    h = _dense_bn(h, wl[...], bl[...], gl[...], bel[...], ones_row)
    h = _dense_bn(h, wm1[...], bm1[...], gm1[...], bem1[...], ones_row)
    h = _dense_bn(h, wm2[...], bm2[...], gm2[...], bem2[...], ones_row)
    out_o[...] = h


def kernel(x, edge_index, batch, W1, b1, W2, b2, Wl, bl, gl, bel,
           Wm1, bm1, gm1, bem1, Wm2, bm2, gm2, bem2):
    del batch
    e = edge_index.shape[1]
    nch = -(-e // (_TILES * _K))
    pad = _TILES * _K * nch - e
    f32 = jnp.float32

    spread = jnp.asarray(
        np.broadcast_to(_N + (np.arange(pad, dtype=np.int32) % (_NP - _N)),
                        (2, pad)))
    both = jnp.concatenate([edge_index, spread], axis=1)
    slabs = both.reshape(2, _TILES, nch, _K)
    srcp = slabs[0]
    dstp = slabs[1]
    xp = jnp.pad(x, ((0, _NP - _N), (0, 0)))

    degp = _deg_kernel(nch)(
        dstp, jnp.ones((_K, 8), f32), jnp.zeros((_NP, 8), f32))

    h1 = pl.pallas_call(
        _tc1a_body, out_shape=jax.ShapeDtypeStruct((_NP, 32), f32))(xp, W1)
    dis, g1 = pl.pallas_call(
        _tc1b_body,
        out_shape=[jax.ShapeDtypeStruct((_NP, 1), f32),
                   jax.ShapeDtypeStruct((_NP, 32), f32)],
    )(degp, h1)

    accp1 = _spmm_kernel(nch, 32, True)(g1, srcp, dstp, jnp.zeros((_NP, 32), f32))

    x1, g2 = pl.pallas_call(
        _tc2_body,
        out_shape=[jax.ShapeDtypeStruct((_NP, 32), f32),
                   jax.ShapeDtypeStruct((_NP, 64), f32)],
    )(accp1, g1, dis, W2, b1.reshape(1, -1))

    accp2 = _spmm_kernel(nch, 64, True)(g2, srcp, dstp, jnp.zeros((_NP, 64), f32))

    out = pl.pallas_call(
        _tc3_body,
        out_shape=jax.ShapeDtypeStruct((_N, 10), f32),
    )(accp2, g2, x1, dis, b2.reshape(1, -1),
      Wl, bl.reshape(1, -1), gl.reshape(1, -1), bel.reshape(1, -1),
      Wm1, bm1.reshape(1, -1), gm1.reshape(1, -1), bem1.reshape(1, -1),
      Wm2, bm2.reshape(1, -1), gm2.reshape(1, -1), bem2.reshape(1, -1))
    return out
